# baseline (speedup 1.0000x reference)
import functools

import jax
import jax.numpy as jnp
from jax import lax
from jax.experimental import pallas as pl
from jax.experimental.pallas import tpu as pltpu
from jax.experimental.pallas import tpu_sc as plsc

N = 100000
E = 1600000
NC, NS = 2, 16
K = 14
KA = 4
ROWS_PAD = 12544
EPAD = ROWS_PAD * 128
RPC = ROWS_PAD // NC
NP = 102400
NA = NP
PR = N * 16 // 128
PRP = NP * 16 // 128
B_TC = 256
GRID = PRP // B_TC

_SC_PARAMS = pltpu.CompilerParams(
    use_tc_tiling_on_sc=False, needs_layout_passes=False)


def _zero_fill(zbuf, rows):
    def body(i, _):
        zbuf[i, :] = jnp.zeros((16,), jnp.float32)
        return 0
    lax.fori_loop(0, rows, body, 0)


def _zero_acc(zbuf, acc, s, zsem):
    def zc(i, _):
        pltpu.async_copy(zbuf, acc.at[pl.ds(i * 128, 128)], zsem)
        return 0
    lax.fori_loop(s * 50, (s + 1) * 50, zc, 0)

    def zw(i, _):
        pltpu.make_async_copy(zbuf, acc.at[pl.ds(0, 128)], zsem).wait()
        return 0
    lax.fori_loop(0, 50, zw, 0)


def _writeback(acc, out_hbm, c, s):
    def wb(i, _):
        off = pl.multiple_of(i * 2048, 8)
        pltpu.sync_copy(acc.at[pl.ds(off, 2048)], out_hbm.at[c, pl.ds(off, 2048)])
        return 0
    lax.fori_loop((s * 50) // 16, ((s + 1) * 50) // 16, wb, 0)


def _rsqrt16(x):
    i = lax.bitcast_convert_type(x, jnp.int32)
    i = jnp.full((16,), 0x5F3759DF, jnp.int32) - lax.shift_right_arithmetic(
        i, jnp.full((16,), 1, jnp.int32))
    y = lax.bitcast_convert_type(i, jnp.float32)
    for _ in range(4):
        y = y * (1.5 - 0.5 * x * y * y)
    return y


@functools.cache
def _sc_front_kernel():
    mesh = plsc.VectorSubcoreMesh(core_axis_name="c", subcore_axis_name="s")
    return pl.kernel(
        _sc_front,
        out_type=(
            jax.ShapeDtypeStruct((NP, 16), jnp.float32),
            jax.ShapeDtypeStruct((NP, 16), jnp.float32),
        ),
        mesh=mesh,
        compiler_params=_SC_PARAMS,
        scratch_types=[
            pltpu.VMEM((2, K, 128), jnp.int32),
            pltpu.VMEM((128,), jnp.float32),
            pltpu.VMEM((2048,), jnp.float32),
            pltpu.VMEM((800,), jnp.float32),
            pltpu.VMEM((800, 16), jnp.float32),
            pltpu.VMEM((800, 16), jnp.float32),
            pltpu.VMEM_SHARED((NA,), jnp.float32),
            pltpu.SemaphoreType.DMA,
            pltpu.SemaphoreType.DMA,
            pltpu.SemaphoreType.DMA,
            pltpu.SemaphoreType.DMA,
        ],
    )


def _sc_front(dst_hbm, xp_hbm, d16_hbm, y1_hbm, dbuf, ones, zbuf, degb, d16b,
              xpb, acc, isem, ssem0, ssem1, zsem):
    c = lax.axis_index("c")
    s = lax.axis_index("s")
    ssems = (ssem0, ssem1)

    def ob(i, _):
        ones[pl.ds(i * 16, 16)] = jnp.full((16,), 1.0, jnp.float32)
        return 0
    lax.fori_loop(0, 8, ob, 0)

    def zb(i, _):
        zbuf[pl.ds(i * 16, 16)] = jnp.zeros((16,), jnp.float32)
        return 0
    lax.fori_loop(0, 128, zb, 0)

    def zc(i, _):
        pltpu.async_copy(zbuf, acc.at[pl.ds(i * 2048, 2048)], zsem)
        return 0
    lax.fori_loop((s * 50) // 16, ((s + 1) * 50) // 16, zc, 0)

    def zw(i, _):
        pltpu.make_async_copy(zbuf, acc.at[pl.ds(0, 2048)], zsem).wait()
        return 0
    lax.fori_loop((s * 50) // 16, ((s + 1) * 50) // 16, zw, 0)
    plsc.subcore_barrier()

    row0 = s * (ROWS_PAD // NS)
    n_chunks = ROWS_PAD // NS // K

    def step(g, slot):
        pltpu.make_async_copy(dst_hbm.at[pl.ds(0, K)], dbuf.at[slot], isem).wait()

        @pl.when(g + 1 < n_chunks)
        def _():
            r = row0 + (g + 1) * K
            pltpu.async_copy(dst_hbm.at[pl.ds(r, K)], dbuf.at[1 - slot], isem)

        @pl.when(g >= 2)
        def _():
            for j in range(K):
                pltpu.make_async_copy(ones, acc.at[dbuf.at[slot, j]], ssems[slot]).wait()

        for j in range(K):
            pltpu.async_copy(ones, acc.at[dbuf.at[slot, j]], ssems[slot], add=True)

    pltpu.async_copy(dst_hbm.at[pl.ds(row0, K)], dbuf.at[0], isem)

    def loop(g2, _):
        step(2 * g2, 0)
        step(2 * g2 + 1, 1)
        return 0
    lax.fori_loop(0, n_chunks // 2, loop, 0)

    for slot in (0, 1):
        for j in range(K):
            pltpu.make_async_copy(ones, acc.at[dbuf.at[slot, j]], ssems[slot]).wait()
    plsc.subcore_barrier()

    wid = c * NS + s
    lane = jnp.arange(16, dtype=jnp.int32)

    if True:
        node0 = wid * 3200

        def chunk(i, _):
            off = node0 + i * 800
            pltpu.sync_copy(acc.at[pl.ds(off, 800)], degb)
            pltpu.sync_copy(xp_hbm.at[pl.ds(off, 800)], xpb)

            def grp(g, _):
                d = degb[pl.ds(g * 16, 16)] + 1.0
                y = _rsqrt16(d)
                row_idx = lane + g * 16
                for l in range(16):
                    plsc.store_scatter(
                        d16b, [row_idx, jnp.full((16,), l, jnp.int32)], y)
                return 0
            lax.fori_loop(0, 50, grp, 0)

            def sc(n, _):
                xpb[n, :] = xpb[n, :] * d16b[n, :]
                return 0
            lax.fori_loop(0, 800, sc, 0)

            pltpu.sync_copy(d16b, d16_hbm.at[pl.ds(off, 800)])
            pltpu.sync_copy(xpb, y1_hbm.at[pl.ds(off, 800)])
            return 0
        lax.fori_loop(0, 4, chunk, 0)


def _agg_pipeline(src_hbm, dst_hbm, table, acc, sbuf, dbuf, rbuf,
                  isem, gsem, ssems, row0, n_chunks):
    def step(g, slot):
        pltpu.make_async_copy(src_hbm.at[pl.ds(0, KA)], sbuf.at[slot], isem).wait()
        pltpu.make_async_copy(dst_hbm.at[pl.ds(0, KA)], dbuf.at[slot], isem).wait()

        @pl.when(g + 1 < n_chunks)
        def _():
            r = row0 + (g + 1) * KA
            pltpu.async_copy(src_hbm.at[pl.ds(r, KA)], sbuf.at[1 - slot], isem)
            pltpu.async_copy(dst_hbm.at[pl.ds(r, KA)], dbuf.at[1 - slot], isem)

        @pl.when(g >= 2)
        def _():
            for j in range(KA):
                pltpu.make_async_copy(
                    rbuf.at[slot, j], acc.at[dbuf.at[slot, j]], ssems[slot]
                ).wait()

        descs = [
            pltpu.async_copy(table.at[sbuf.at[slot, j]], rbuf.at[slot, j], gsem)
            for j in range(KA)
        ]
        for d in descs:
            d.wait()
        for j in range(KA):
            pltpu.async_copy(
                rbuf.at[slot, j], acc.at[dbuf.at[slot, j]], ssems[slot], add=True
            )

    pltpu.async_copy(src_hbm.at[pl.ds(row0, KA)], sbuf.at[0], isem)
    pltpu.async_copy(dst_hbm.at[pl.ds(row0, KA)], dbuf.at[0], isem)

    def loop(g2, _):
        step(2 * g2, 0)
        step(2 * g2 + 1, 1)
        return 0
    lax.fori_loop(0, n_chunks // 2, loop, 0)

    for slot in (0, 1):
        for j in range(KA):
            pltpu.make_async_copy(
                rbuf.at[slot, j], acc.at[dbuf.at[slot, j]], ssems[slot]
            ).wait()


_AGG_SCRATCH = [
    pltpu.VMEM((2, KA, 128), jnp.int32),
    pltpu.VMEM((2, KA, 128), jnp.int32),
    pltpu.VMEM((2, KA, 128, 16), jnp.float32),
    pltpu.VMEM((128, 16), jnp.float32),
    pltpu.VMEM_SHARED((NP, 16), jnp.float32),
    pltpu.SemaphoreType.DMA,
    pltpu.SemaphoreType.DMA,
    pltpu.SemaphoreType.DMA,
    pltpu.SemaphoreType.DMA,
    pltpu.SemaphoreType.DMA,
]


@functools.cache
def _sc_agg1_kernel():
    mesh = plsc.VectorSubcoreMesh(core_axis_name="c", subcore_axis_name="s")
    return pl.kernel(
        _sc_agg1,
        out_type=jax.ShapeDtypeStruct((NC, NP, 16), jnp.float32),
        mesh=mesh,
        compiler_params=_SC_PARAMS,
        scratch_types=list(_AGG_SCRATCH),
    )


def _sc_agg1(src_hbm, dst_hbm, y1_hbm, out_hbm, sbuf, dbuf, rbuf, zbuf, acc,
             isem, gsem, ssem0, ssem1, zsem):
    c = lax.axis_index("c")
    s = lax.axis_index("s")
    _zero_fill(zbuf, 128)
    _zero_acc(zbuf, acc, s, zsem)
    plsc.subcore_barrier()

    row0 = c * RPC + s * (RPC // NS)
    _agg_pipeline(src_hbm, dst_hbm, y1_hbm, acc, sbuf, dbuf, rbuf,
                  isem, gsem, (ssem0, ssem1), row0, (RPC // NS) // KA)
    plsc.subcore_barrier()
    _writeback(acc, out_hbm, c, s)


@functools.cache
def _sc_agg2_kernel():
    mesh = plsc.VectorSubcoreMesh(core_axis_name="c", subcore_axis_name="s")
    return pl.kernel(
        _sc_agg2,
        out_type=jax.ShapeDtypeStruct((NC, NP, 16), jnp.float32),
        mesh=mesh,
        compiler_params=_SC_PARAMS,
        scratch_types=list(_AGG_SCRATCH),
    )


def _sc_agg2(srca_hbm, srcb_hbm, dst_hbm, y2pairs_hbm, out_hbm, sbuf, dbuf,
             rbuf, zbuf, acc, isem, gsem, ssem0, ssem1, zsem):
    c = lax.axis_index("c")
    s = lax.axis_index("s")
    _zero_fill(zbuf, 128)
    _zero_acc(zbuf, acc, s, zsem)
    plsc.subcore_barrier()

    row0 = s * (ROWS_PAD // NS)
    n_chunks = ROWS_PAD // NS // KA

    @pl.when(c == 0)
    def _():
        _agg_pipeline(srca_hbm, dst_hbm, y2pairs_hbm, acc, sbuf, dbuf, rbuf,
                      isem, gsem, (ssem0, ssem1), row0, n_chunks)

    @pl.when(c == 1)
    def _():
        _agg_pipeline(srcb_hbm, dst_hbm, y2pairs_hbm, acc, sbuf, dbuf, rbuf,
                      isem, gsem, (ssem0, ssem1), row0, n_chunks)

    plsc.subcore_barrier()
    _writeback(acc, out_hbm, c, s)


def _tc2_body(aggp_ref, y1_ref, d16_ref, w1b_ref, b1b_ref, w2b_ref, e32_ref,
              y2p_ref):
    d16 = d16_ref[...]
    pp = (aggp_ref[0] + aggp_ref[1] + y1_ref[...]) * d16
    h1 = jnp.maximum(
        jnp.dot(pp, w1b_ref[...], preferred_element_type=jnp.float32)
        + b1b_ref[...],
        0.0,
    )
    d32 = jnp.dot(d16, e32_ref[...], preferred_element_type=jnp.float32)
    y2p_ref[...] = jnp.dot(
        h1, w2b_ref[...], preferred_element_type=jnp.float32) * d32


def _tc2(agg1p, y1p, d16p, w1blk, b1blk, w2blk, e32):
    return pl.pallas_call(
        _tc2_body,
        grid=(GRID,),
        in_specs=[
            pl.BlockSpec((2, B_TC, 128), lambda i: (0, i, 0)),
            pl.BlockSpec((B_TC, 128), lambda i: (i, 0)),
            pl.BlockSpec((B_TC, 128), lambda i: (i, 0)),
            pl.BlockSpec((128, 512), lambda i: (0, 0)),
            pl.BlockSpec((1, 512), lambda i: (0, 0)),
            pl.BlockSpec((512, 256), lambda i: (0, 0)),
            pl.BlockSpec((128, 256), lambda i: (0, 0)),
        ],
        out_specs=pl.BlockSpec((B_TC, 256), lambda i: (i, 0)),
        out_shape=jax.ShapeDtypeStruct((PRP, 256), jnp.float32),
    )(agg1p, y1p, d16p, w1blk, b1blk, w2blk, e32)


def _tc3_body(agg32_ref, y2p_ref, d16_ref, e32_ref, b2t_ref, wout_ref,
              bout_ref, out_ref, sacc_ref):
    i = pl.program_id(0)
    d32 = jnp.dot(d16_ref[...], e32_ref[...],
                  preferred_element_type=jnp.float32)
    h2 = jnp.maximum(
        (agg32_ref[...] + y2p_ref[...]) * d32 + b2t_ref[...], 0.0)
    rows = i * B_TC + lax.broadcasted_iota(jnp.int32, (B_TC, 256), 0)
    h2 = jnp.where(rows < PR, h2, 0.0)
    ps = jnp.sum(h2, axis=0, keepdims=True)

    @pl.when(i == 0)
    def _():
        sacc_ref[...] = ps

    @pl.when(i > 0)
    def _():
        sacc_ref[...] = sacc_ref[...] + ps

    @pl.when(i == GRID - 1)
    def _():
        fold = jnp.asarray(
            lax.broadcasted_iota(jnp.int32, (256, 32), 0) % 32
            == lax.broadcasted_iota(jnp.int32, (256, 32), 1),
            dtype=jnp.float32,
        )
        g = jnp.dot(sacc_ref[...] * (1.0 / N), fold,
                    preferred_element_type=jnp.float32)
        t = jnp.dot(g, wout_ref[...],
                    preferred_element_type=jnp.float32) + bout_ref[...]
        out_ref[...] = 1.0 / (1.0 + jnp.exp(-t))


def _tc3(agg32, y2p, d16p, e32, b2t, wout, boutr):
    return pl.pallas_call(
        _tc3_body,
        grid=(GRID,),
        in_specs=[
            pl.BlockSpec((B_TC, 256), lambda i: (i, 0)),
            pl.BlockSpec((B_TC, 256), lambda i: (i, 0)),
            pl.BlockSpec((B_TC, 128), lambda i: (i, 0)),
            pl.BlockSpec((128, 256), lambda i: (0, 0)),
            pl.BlockSpec((1, 256), lambda i: (0, 0)),
            pl.BlockSpec((32, 1), lambda i: (0, 0)),
            pl.BlockSpec((1, 1), lambda i: (0, 0)),
        ],
        out_specs=pl.BlockSpec((1, 1), lambda i: (0, 0)),
        out_shape=jax.ShapeDtypeStruct((1, 1), jnp.float32),
        scratch_shapes=[pltpu.VMEM((1, 256), jnp.float32)],
    )(agg32, y2p, d16p, e32, b2t, wout, boutr)


@jax.jit
def kernel(x, edge_index, W1, b1, W2, b2, Wout, bout):
    pad = EPAD - E
    src = jnp.concatenate([edge_index[0], jnp.zeros((pad,), jnp.int32)])
    dst2d = jnp.concatenate(
        [edge_index[1], jnp.full((pad,), N, jnp.int32)]).reshape(ROWS_PAD, 128)
    src2d = src.reshape(ROWS_PAD, 128)
    src2a = (src * 2).reshape(ROWS_PAD, 128)
    src2b = (src * 2 + 1).reshape(ROWS_PAD, 128)
    xp = jnp.pad(x, ((0, NP - N), (0, 16 - x.shape[1])))
    w1p = jnp.pad(W1, ((0, 16 - W1.shape[0]), (0, 0)))
    eye8 = jnp.eye(8, dtype=jnp.float32)
    w1blk = jnp.kron(eye8, w1p)
    w2blk = jnp.kron(eye8, W2)
    e32 = jnp.kron(eye8, jnp.full((16, 32), 1.0 / 16.0))
    b1blk = jnp.tile(b1, 8).reshape(1, 512)
    b2t = jnp.tile(b2, 8).reshape(1, 256)

    d16, y1 = _sc_front_kernel()(dst2d, xp)
    agg1p = _sc_agg1_kernel()(src2d, dst2d, y1)
    y2p = _tc2(
        agg1p.reshape(NC, PRP, 128), y1.reshape(PRP, 128), d16.reshape(PRP, 128),
        w1blk, b1blk, w2blk, e32)
    y2pairs = y2p.reshape(2 * NP, 16)
    agg2p = _sc_agg2_kernel()(src2a, src2b, dst2d, y2pairs)
    agg32 = jnp.concatenate(
        [agg2p[0], agg2p[1]], axis=1).reshape(PRP, 256)
    out = _tc3(agg32, y2p, d16.reshape(PRP, 128), e32, b2t,
               Wout, bout.reshape(1, 1))
    return out.reshape(1)


# speedup vs baseline: 1.2572x; 1.2572x over previous
import functools

import jax
import jax.numpy as jnp
from jax import lax
from jax.experimental import pallas as pl
from jax.experimental.pallas import tpu as pltpu
from jax.experimental.pallas import tpu_sc as plsc

N = 100000
E = 1600000
NC, NS = 2, 16
K = 14
KA = 4
ROWS_PAD = 12544
EPAD = ROWS_PAD * 128
RPC = ROWS_PAD // NC
NP = 102400
NA = NP
PR = N * 16 // 128
PRP = NP * 16 // 128
B_TC = 256
GRID = PRP // B_TC

_SC_PARAMS = pltpu.CompilerParams(use_tc_tiling_on_sc=False)
_SC_PARAMS_NLP = pltpu.CompilerParams(
    use_tc_tiling_on_sc=False, needs_layout_passes=False)


def _zero_fill(zbuf, rows):
    def body(i, _):
        zbuf[i, :] = jnp.zeros((16,), jnp.float32)
        return 0
    lax.fori_loop(0, rows, body, 0)


def _zero_acc(zbuf, acc, s, zsem):
    def zc(i, _):
        pltpu.async_copy(zbuf, acc.at[pl.ds(i * 128, 128)], zsem)
        return 0
    lax.fori_loop(s * 50, (s + 1) * 50, zc, 0)

    def zw(i, _):
        pltpu.make_async_copy(zbuf, acc.at[pl.ds(0, 128)], zsem).wait()
        return 0
    lax.fori_loop(0, 50, zw, 0)


def _writeback(acc, out_hbm, c, s):
    def wb(i, _):
        off = pl.multiple_of(i * 2048, 8)
        pltpu.sync_copy(acc.at[pl.ds(off, 2048)], out_hbm.at[c, pl.ds(off, 2048)])
        return 0
    lax.fori_loop((s * 50) // 16, ((s + 1) * 50) // 16, wb, 0)


def _rsqrt16(x):
    i = lax.bitcast_convert_type(x, jnp.int32)
    i = jnp.full((16,), 0x5F3759DF, jnp.int32) - lax.shift_right_arithmetic(
        i, jnp.full((16,), 1, jnp.int32))
    y = lax.bitcast_convert_type(i, jnp.float32)
    for _ in range(4):
        y = y * (1.5 - 0.5 * x * y * y)
    return y


@functools.cache
def _sc_front_kernel():
    mesh = plsc.VectorSubcoreMesh(core_axis_name="c", subcore_axis_name="s")
    return pl.kernel(
        _sc_front,
        out_type=(
            jax.ShapeDtypeStruct((NP, 16), jnp.float32),
            jax.ShapeDtypeStruct((NP, 16), jnp.float32),
        ),
        mesh=mesh,
        compiler_params=_SC_PARAMS_NLP,
        scratch_types=[
            pltpu.VMEM((2, K, 128), jnp.int32),
            pltpu.VMEM((128,), jnp.float32),
            pltpu.VMEM((2048,), jnp.float32),
            pltpu.VMEM((800,), jnp.float32),
            pltpu.VMEM((800, 16), jnp.float32),
            pltpu.VMEM((800, 16), jnp.float32),
            pltpu.VMEM_SHARED((NA,), jnp.float32),
            pltpu.SemaphoreType.DMA,
            pltpu.SemaphoreType.DMA,
            pltpu.SemaphoreType.DMA,
            pltpu.SemaphoreType.DMA,
        ],
    )


def _sc_front(dst_hbm, xp_hbm, d16_hbm, y1_hbm, dbuf, ones, zbuf, degb, d16b,
              xpb, acc, isem, ssem0, ssem1, zsem):
    c = lax.axis_index("c")
    s = lax.axis_index("s")
    ssems = (ssem0, ssem1)

    def ob(i, _):
        ones[pl.ds(i * 16, 16)] = jnp.full((16,), 1.0, jnp.float32)
        return 0
    lax.fori_loop(0, 8, ob, 0)

    def zb(i, _):
        zbuf[pl.ds(i * 16, 16)] = jnp.zeros((16,), jnp.float32)
        return 0
    lax.fori_loop(0, 128, zb, 0)

    def zc(i, _):
        pltpu.async_copy(zbuf, acc.at[pl.ds(i * 2048, 2048)], zsem)
        return 0
    lax.fori_loop((s * 50) // 16, ((s + 1) * 50) // 16, zc, 0)

    def zw(i, _):
        pltpu.make_async_copy(zbuf, acc.at[pl.ds(0, 2048)], zsem).wait()
        return 0
    lax.fori_loop((s * 50) // 16, ((s + 1) * 50) // 16, zw, 0)
    plsc.subcore_barrier()

    row0 = s * (ROWS_PAD // NS)
    n_chunks = ROWS_PAD // NS // K

    def step(g, slot):
        pltpu.make_async_copy(dst_hbm.at[pl.ds(0, K)], dbuf.at[slot], isem).wait()

        @pl.when(g + 1 < n_chunks)
        def _():
            r = row0 + (g + 1) * K
            pltpu.async_copy(dst_hbm.at[pl.ds(r, K)], dbuf.at[1 - slot], isem)

        @pl.when(g >= 2)
        def _():
            for j in range(K):
                pltpu.make_async_copy(ones, acc.at[dbuf.at[slot, j]], ssems[slot]).wait()

        for j in range(K):
            pltpu.async_copy(ones, acc.at[dbuf.at[slot, j]], ssems[slot], add=True)

    pltpu.async_copy(dst_hbm.at[pl.ds(row0, K)], dbuf.at[0], isem)

    def loop(g2, _):
        step(2 * g2, 0)
        step(2 * g2 + 1, 1)
        return 0
    lax.fori_loop(0, n_chunks // 2, loop, 0)

    for slot in (0, 1):
        for j in range(K):
            pltpu.make_async_copy(ones, acc.at[dbuf.at[slot, j]], ssems[slot]).wait()
    plsc.subcore_barrier()

    wid = c * NS + s
    lane = jnp.arange(16, dtype=jnp.int32)

    if True:
        node0 = wid * 3200

        def chunk(i, _):
            off = node0 + i * 800
            pltpu.sync_copy(acc.at[pl.ds(off, 800)], degb)
            pltpu.sync_copy(xp_hbm.at[pl.ds(off, 800)], xpb)

            def grp(g, _):
                d = degb[pl.ds(g * 16, 16)] + 1.0
                y = _rsqrt16(d)
                row_idx = lane + g * 16
                for l in range(16):
                    plsc.store_scatter(
                        d16b, [row_idx, jnp.full((16,), l, jnp.int32)], y)
                return 0
            lax.fori_loop(0, 50, grp, 0)

            def sc(n, _):
                xpb[n, :] = xpb[n, :] * d16b[n, :]
                return 0
            lax.fori_loop(0, 800, sc, 0)

            pltpu.sync_copy(d16b, d16_hbm.at[pl.ds(off, 800)])
            pltpu.sync_copy(xpb, y1_hbm.at[pl.ds(off, 800)])
            return 0
        lax.fori_loop(0, 4, chunk, 0)


def _agg_pipeline(src_hbm, dst_hbm, table, acc, sbuf, dbuf, rbuf,
                  isem, gsem, ssems, row0, n_chunks):
    def step(g, slot):
        pltpu.make_async_copy(src_hbm.at[pl.ds(0, KA)], sbuf.at[slot], isem).wait()
        pltpu.make_async_copy(dst_hbm.at[pl.ds(0, KA)], dbuf.at[slot], isem).wait()

        @pl.when(g + 1 < n_chunks)
        def _():
            r = row0 + (g + 1) * KA
            pltpu.async_copy(src_hbm.at[pl.ds(r, KA)], sbuf.at[1 - slot], isem)
            pltpu.async_copy(dst_hbm.at[pl.ds(r, KA)], dbuf.at[1 - slot], isem)

        @pl.when(g >= 2)
        def _():
            for j in range(KA):
                pltpu.make_async_copy(
                    rbuf.at[slot, j], acc.at[dbuf.at[slot, j]], ssems[slot]
                ).wait()

        descs = [
            pltpu.async_copy(table.at[sbuf.at[slot, j]], rbuf.at[slot, j], gsem)
            for j in range(KA)
        ]
        for d in descs:
            d.wait()
        for j in range(KA):
            pltpu.async_copy(
                rbuf.at[slot, j], acc.at[dbuf.at[slot, j]], ssems[slot], add=True
            )

    pltpu.async_copy(src_hbm.at[pl.ds(row0, KA)], sbuf.at[0], isem)
    pltpu.async_copy(dst_hbm.at[pl.ds(row0, KA)], dbuf.at[0], isem)

    def loop(g2, _):
        step(2 * g2, 0)
        step(2 * g2 + 1, 1)
        return 0
    lax.fori_loop(0, n_chunks // 2, loop, 0)

    for slot in (0, 1):
        for j in range(KA):
            pltpu.make_async_copy(
                rbuf.at[slot, j], acc.at[dbuf.at[slot, j]], ssems[slot]
            ).wait()


_AGG_SCRATCH = [
    pltpu.VMEM((2, KA, 128), jnp.int32),
    pltpu.VMEM((2, KA, 128), jnp.int32),
    pltpu.VMEM((2, KA, 128, 16), jnp.float32),
    pltpu.VMEM((128, 16), jnp.float32),
    pltpu.VMEM_SHARED((NP, 16), jnp.float32),
    pltpu.SemaphoreType.DMA,
    pltpu.SemaphoreType.DMA,
    pltpu.SemaphoreType.DMA,
    pltpu.SemaphoreType.DMA,
    pltpu.SemaphoreType.DMA,
]


@functools.cache
def _sc_agg1_kernel():
    mesh = plsc.VectorSubcoreMesh(core_axis_name="c", subcore_axis_name="s")
    return pl.kernel(
        _sc_agg1,
        out_type=jax.ShapeDtypeStruct((NC, NP, 16), jnp.float32),
        mesh=mesh,
        compiler_params=_SC_PARAMS,
        scratch_types=list(_AGG_SCRATCH),
    )


def _sc_agg1(src_hbm, dst_hbm, y1_hbm, out_hbm, sbuf, dbuf, rbuf, zbuf, acc,
             isem, gsem, ssem0, ssem1, zsem):
    c = lax.axis_index("c")
    s = lax.axis_index("s")
    _zero_fill(zbuf, 128)
    _zero_acc(zbuf, acc, s, zsem)
    plsc.subcore_barrier()

    row0 = c * RPC + s * (RPC // NS)
    _agg_pipeline(src_hbm, dst_hbm, y1_hbm, acc, sbuf, dbuf, rbuf,
                  isem, gsem, (ssem0, ssem1), row0, (RPC // NS) // KA)
    plsc.subcore_barrier()
    _writeback(acc, out_hbm, c, s)


@functools.cache
def _sc_agg2_kernel():
    mesh = plsc.VectorSubcoreMesh(core_axis_name="c", subcore_axis_name="s")
    return pl.kernel(
        _sc_agg2,
        out_type=jax.ShapeDtypeStruct((NC, NP, 16), jnp.float32),
        mesh=mesh,
        compiler_params=_SC_PARAMS,
        scratch_types=list(_AGG_SCRATCH),
    )


def _sc_agg2(srca_hbm, srcb_hbm, dst_hbm, y2pairs_hbm, out_hbm, sbuf, dbuf,
             rbuf, zbuf, acc, isem, gsem, ssem0, ssem1, zsem):
    c = lax.axis_index("c")
    s = lax.axis_index("s")
    _zero_fill(zbuf, 128)
    _zero_acc(zbuf, acc, s, zsem)
    plsc.subcore_barrier()

    row0 = s * (ROWS_PAD // NS)
    n_chunks = ROWS_PAD // NS // KA

    @pl.when(c == 0)
    def _():
        _agg_pipeline(srca_hbm, dst_hbm, y2pairs_hbm, acc, sbuf, dbuf, rbuf,
                      isem, gsem, (ssem0, ssem1), row0, n_chunks)

    @pl.when(c == 1)
    def _():
        _agg_pipeline(srcb_hbm, dst_hbm, y2pairs_hbm, acc, sbuf, dbuf, rbuf,
                      isem, gsem, (ssem0, ssem1), row0, n_chunks)

    plsc.subcore_barrier()
    _writeback(acc, out_hbm, c, s)


def _tc2_body(aggp_ref, y1_ref, d16_ref, w1b_ref, b1b_ref, w2b_ref, e32_ref,
              y2p_ref):
    d16 = d16_ref[...]
    pp = (aggp_ref[0] + aggp_ref[1] + y1_ref[...]) * d16
    h1 = jnp.maximum(
        jnp.dot(pp, w1b_ref[...], preferred_element_type=jnp.float32)
        + b1b_ref[...],
        0.0,
    )
    d32 = jnp.dot(d16, e32_ref[...], preferred_element_type=jnp.float32)
    y2p_ref[...] = jnp.dot(
        h1, w2b_ref[...], preferred_element_type=jnp.float32) * d32


def _tc2(agg1p, y1p, d16p, w1blk, b1blk, w2blk, e32):
    return pl.pallas_call(
        _tc2_body,
        grid=(GRID,),
        in_specs=[
            pl.BlockSpec((2, B_TC, 128), lambda i: (0, i, 0)),
            pl.BlockSpec((B_TC, 128), lambda i: (i, 0)),
            pl.BlockSpec((B_TC, 128), lambda i: (i, 0)),
            pl.BlockSpec((128, 512), lambda i: (0, 0)),
            pl.BlockSpec((1, 512), lambda i: (0, 0)),
            pl.BlockSpec((512, 256), lambda i: (0, 0)),
            pl.BlockSpec((128, 256), lambda i: (0, 0)),
        ],
        out_specs=pl.BlockSpec((B_TC, 256), lambda i: (i, 0)),
        out_shape=jax.ShapeDtypeStruct((PRP, 256), jnp.float32),
    )(agg1p, y1p, d16p, w1blk, b1blk, w2blk, e32)


def _tc3_body(aggp_ref, y2p_ref, d16_ref, e32_ref, sa_ref, sb_ref, b2t_ref,
              wout_ref, bout_ref, out_ref, sacc_ref):
    i = pl.program_id(0)
    d32 = jnp.dot(d16_ref[...], e32_ref[...],
                  preferred_element_type=jnp.float32)
    agg32 = (jnp.dot(aggp_ref[0], sa_ref[...], preferred_element_type=jnp.float32)
             + jnp.dot(aggp_ref[1], sb_ref[...], preferred_element_type=jnp.float32))
    h2 = jnp.maximum(
        (agg32 + y2p_ref[...]) * d32 + b2t_ref[...], 0.0)
    rows = i * B_TC + lax.broadcasted_iota(jnp.int32, (B_TC, 256), 0)
    h2 = jnp.where(rows < PR, h2, 0.0)
    ps = jnp.sum(h2, axis=0, keepdims=True)

    @pl.when(i == 0)
    def _():
        sacc_ref[...] = ps

    @pl.when(i > 0)
    def _():
        sacc_ref[...] = sacc_ref[...] + ps

    @pl.when(i == GRID - 1)
    def _():
        fold = jnp.asarray(
            lax.broadcasted_iota(jnp.int32, (256, 32), 0) % 32
            == lax.broadcasted_iota(jnp.int32, (256, 32), 1),
            dtype=jnp.float32,
        )
        g = jnp.dot(sacc_ref[...] * (1.0 / N), fold,
                    preferred_element_type=jnp.float32)
        t = jnp.dot(g, wout_ref[...],
                    preferred_element_type=jnp.float32) + bout_ref[...]
        out_ref[...] = 1.0 / (1.0 + jnp.exp(-t))


def _tc3(agg2p, y2p, d16p, e32, sa, sb, b2t, wout, boutr):
    return pl.pallas_call(
        _tc3_body,
        grid=(GRID,),
        in_specs=[
            pl.BlockSpec((2, B_TC, 128), lambda i: (0, i, 0)),
            pl.BlockSpec((B_TC, 256), lambda i: (i, 0)),
            pl.BlockSpec((B_TC, 128), lambda i: (i, 0)),
            pl.BlockSpec((128, 256), lambda i: (0, 0)),
            pl.BlockSpec((128, 256), lambda i: (0, 0)),
            pl.BlockSpec((128, 256), lambda i: (0, 0)),
            pl.BlockSpec((1, 256), lambda i: (0, 0)),
            pl.BlockSpec((32, 1), lambda i: (0, 0)),
            pl.BlockSpec((1, 1), lambda i: (0, 0)),
        ],
        out_specs=pl.BlockSpec((1, 1), lambda i: (0, 0)),
        out_shape=jax.ShapeDtypeStruct((1, 1), jnp.float32),
        scratch_shapes=[pltpu.VMEM((1, 256), jnp.float32)],
    )(agg2p, y2p, d16p, e32, sa, sb, b2t, wout, boutr)


@jax.jit
def kernel(x, edge_index, W1, b1, W2, b2, Wout, bout):
    pad = EPAD - E
    src = jnp.concatenate([edge_index[0], jnp.zeros((pad,), jnp.int32)])
    dst2d = jnp.concatenate(
        [edge_index[1], jnp.full((pad,), N, jnp.int32)]).reshape(ROWS_PAD, 128)
    src2d = src.reshape(ROWS_PAD, 128)
    src2a = (src * 2).reshape(ROWS_PAD, 128)
    src2b = (src * 2 + 1).reshape(ROWS_PAD, 128)
    xp = jnp.pad(x, ((0, NP - N), (0, 16 - x.shape[1])))
    w1p = jnp.pad(W1, ((0, 16 - W1.shape[0]), (0, 0)))
    eye8 = jnp.eye(8, dtype=jnp.float32)
    w1blk = jnp.kron(eye8, w1p)
    w2blk = jnp.kron(eye8, W2)
    e32 = jnp.kron(eye8, jnp.full((16, 32), 1.0 / 16.0))
    b1blk = jnp.tile(b1, 8).reshape(1, 512)
    b2t = jnp.tile(b2, 8).reshape(1, 256)

    d16, y1 = _sc_front_kernel()(dst2d, xp)
    agg1p = _sc_agg1_kernel()(src2d, dst2d, y1)
    y2p = _tc2(
        agg1p.reshape(NC, PRP, 128), y1.reshape(PRP, 128), d16.reshape(PRP, 128),
        w1blk, b1blk, w2blk, e32)
    y2pairs = y2p.reshape(2 * NP, 16)
    agg2p = _sc_agg2_kernel()(src2a, src2b, dst2d, y2pairs)
    i16 = jnp.eye(16, dtype=jnp.float32)
    z16 = jnp.zeros((16, 16), jnp.float32)
    sa = jnp.kron(eye8, jnp.concatenate([i16, z16], axis=1))
    sb = jnp.kron(eye8, jnp.concatenate([z16, i16], axis=1))
    out = _tc3(agg2p.reshape(NC, PRP, 128), y2p, d16.reshape(PRP, 128),
               e32, sa, sb, b2t, Wout, bout.reshape(1, 1))
    return out.reshape(1)


# speedup vs baseline: 1.4748x; 1.1731x over previous
import functools

import jax
import jax.numpy as jnp
from jax import lax
from jax.experimental import pallas as pl
from jax.experimental.pallas import tpu as pltpu
from jax.experimental.pallas import tpu_sc as plsc

N = 100000
E = 1600000
NC, NS = 2, 16
K = 14
KA = 4
ROWS_PAD = 12544
EPAD = ROWS_PAD * 128
RPC = ROWS_PAD // NC
NP = 102400
NA = NP
PR = N * 16 // 128
PRP = NP * 16 // 128
B_TC = 256
GRID = PRP // B_TC

_SC_PARAMS = pltpu.CompilerParams(use_tc_tiling_on_sc=False)
_SC_PARAMS_NLP = pltpu.CompilerParams(
    use_tc_tiling_on_sc=False, needs_layout_passes=False)


def _zero_fill(zbuf, rows):
    def body(i, _):
        zbuf[i, :] = jnp.zeros((16,), jnp.float32)
        return 0
    lax.fori_loop(0, rows, body, 0)


def _zero_acc(zbuf, acc, s, zsem):
    def zc(i, _):
        pltpu.async_copy(zbuf, acc.at[pl.ds(i * 128, 128)], zsem)
        return 0
    lax.fori_loop(s * 50, (s + 1) * 50, zc, 0)

    def zw(i, _):
        pltpu.make_async_copy(zbuf, acc.at[pl.ds(0, 128)], zsem).wait()
        return 0
    lax.fori_loop(0, 50, zw, 0)


def _writeback(acc, out_hbm, c, s):
    def wb(i, _):
        off = pl.multiple_of(i * 2048, 8)
        pltpu.sync_copy(acc.at[pl.ds(off, 2048)], out_hbm.at[c, pl.ds(off, 2048)])
        return 0
    lax.fori_loop((s * 50) // 16, ((s + 1) * 50) // 16, wb, 0)


def _rsqrt16(x):
    i = lax.bitcast_convert_type(x, jnp.int32)
    i = jnp.full((16,), 0x5F3759DF, jnp.int32) - lax.shift_right_arithmetic(
        i, jnp.full((16,), 1, jnp.int32))
    y = lax.bitcast_convert_type(i, jnp.float32)
    for _ in range(4):
        y = y * (1.5 - 0.5 * x * y * y)
    return y


@functools.cache
def _sc_front_kernel():
    mesh = plsc.VectorSubcoreMesh(core_axis_name="c", subcore_axis_name="s")
    return pl.kernel(
        _sc_front,
        out_type=(
            jax.ShapeDtypeStruct((NP, 16), jnp.float32),
            jax.ShapeDtypeStruct((NP, 16), jnp.float32),
        ),
        mesh=mesh,
        compiler_params=_SC_PARAMS_NLP,
        scratch_types=[
            pltpu.VMEM((2, K, 128), jnp.int32),
            pltpu.VMEM((128,), jnp.float32),
            pltpu.VMEM((2048,), jnp.float32),
            pltpu.VMEM((800,), jnp.float32),
            pltpu.VMEM((800, 16), jnp.float32),
            pltpu.VMEM((800, 16), jnp.float32),
            pltpu.VMEM_SHARED((NA,), jnp.float32),
            pltpu.SemaphoreType.DMA,
            pltpu.SemaphoreType.DMA,
            pltpu.SemaphoreType.DMA,
            pltpu.SemaphoreType.DMA,
        ],
    )


def _sc_front(dst_hbm, xp_hbm, d16_hbm, y1_hbm, dbuf, ones, zbuf, degb, d16b,
              xpb, acc, isem, ssem0, ssem1, zsem):
    c = lax.axis_index("c")
    s = lax.axis_index("s")
    ssems = (ssem0, ssem1)

    def ob(i, _):
        ones[pl.ds(i * 16, 16)] = jnp.full((16,), 1.0, jnp.float32)
        return 0
    lax.fori_loop(0, 8, ob, 0)

    def zb(i, _):
        zbuf[pl.ds(i * 16, 16)] = jnp.zeros((16,), jnp.float32)
        return 0
    lax.fori_loop(0, 128, zb, 0)

    def zc(i, _):
        pltpu.async_copy(zbuf, acc.at[pl.ds(i * 2048, 2048)], zsem)
        return 0
    lax.fori_loop((s * 50) // 16, ((s + 1) * 50) // 16, zc, 0)

    def zw(i, _):
        pltpu.make_async_copy(zbuf, acc.at[pl.ds(0, 2048)], zsem).wait()
        return 0
    lax.fori_loop((s * 50) // 16, ((s + 1) * 50) // 16, zw, 0)
    plsc.subcore_barrier()

    row0 = s * (ROWS_PAD // NS)
    n_chunks = ROWS_PAD // NS // K

    def step(g, slot):
        pltpu.make_async_copy(dst_hbm.at[pl.ds(0, K)], dbuf.at[slot], isem).wait()

        @pl.when(g + 1 < n_chunks)
        def _():
            r = row0 + (g + 1) * K
            pltpu.async_copy(dst_hbm.at[pl.ds(r, K)], dbuf.at[1 - slot], isem)

        @pl.when(g >= 2)
        def _():
            for j in range(K):
                pltpu.make_async_copy(ones, acc.at[dbuf.at[slot, j]], ssems[slot]).wait()

        for j in range(K):
            pltpu.async_copy(ones, acc.at[dbuf.at[slot, j]], ssems[slot], add=True)

    pltpu.async_copy(dst_hbm.at[pl.ds(row0, K)], dbuf.at[0], isem)

    def loop(g2, _):
        step(2 * g2, 0)
        step(2 * g2 + 1, 1)
        return 0
    lax.fori_loop(0, n_chunks // 2, loop, 0)

    for slot in (0, 1):
        for j in range(K):
            pltpu.make_async_copy(ones, acc.at[dbuf.at[slot, j]], ssems[slot]).wait()
    plsc.subcore_barrier()

    wid = c * NS + s
    lane = jnp.arange(16, dtype=jnp.int32)

    if True:
        node0 = wid * 3200

        def chunk(i, _):
            off = node0 + i * 800
            pltpu.sync_copy(acc.at[pl.ds(off, 800)], degb)
            pltpu.sync_copy(xp_hbm.at[pl.ds(off, 800)], xpb)

            def grp(g, _):
                d = degb[pl.ds(g * 16, 16)] + 1.0
                y = _rsqrt16(d)
                row_idx = lane + g * 16
                for l in range(16):
                    plsc.store_scatter(
                        d16b, [row_idx, jnp.full((16,), l, jnp.int32)], y)
                return 0
            lax.fori_loop(0, 50, grp, 0)

            def sc(n, _):
                xpb[n, :] = xpb[n, :] * d16b[n, :]
                return 0
            lax.fori_loop(0, 800, sc, 0)

            pltpu.sync_copy(d16b, d16_hbm.at[pl.ds(off, 800)])
            pltpu.sync_copy(xpb, y1_hbm.at[pl.ds(off, 800)])
            return 0
        lax.fori_loop(0, 4, chunk, 0)


def _agg_pipeline(src_hbm, dst_hbm, table, acc, sbuf, dbuf, rbuf,
                  isems, gsems, ssems, row0, n_chunks):
    def idx_fetch(g, islot, par):
        r = row0 + g * KA
        pltpu.async_copy(src_hbm.at[pl.ds(r, KA)], sbuf.at[islot], isems[par])
        pltpu.async_copy(dst_hbm.at[pl.ds(r, KA)], dbuf.at[islot], isems[par])

    def idx_wait(islot, par):
        pltpu.make_async_copy(src_hbm.at[pl.ds(0, KA)], sbuf.at[islot], isems[par]).wait()
        pltpu.make_async_copy(dst_hbm.at[pl.ds(0, KA)], dbuf.at[islot], isems[par]).wait()

    def scat_fire(rslot, islot, par):
        for j in range(KA):
            pltpu.async_copy(rbuf.at[rslot, j], acc.at[dbuf.at[islot, j]],
                             ssems[par], add=True)

    def scat_drain(rslot, islot, par):
        for j in range(KA):
            pltpu.make_async_copy(rbuf.at[rslot, j], acc.at[dbuf.at[islot, j]],
                                  ssems[par]).wait()

    def gath_fire(rslot, islot, par):
        for j in range(KA):
            pltpu.async_copy(table.at[sbuf.at[islot, j]], rbuf.at[rslot, j],
                             gsems[par])

    def gath_drain(rslot, islot, par):
        for j in range(KA):
            pltpu.make_async_copy(table.at[sbuf.at[islot, j]], rbuf.at[rslot, j],
                                  gsems[par]).wait()

    def step(g, g2, par):
        islot = lax.rem(g2, 2) * 2 + par
        idx_wait(islot, par)

        @pl.when(g >= 2)
        def _():
            scat_drain(par, islot, par)

        gath_fire(par, islot, par)

        @pl.when(g + 2 < n_chunks)
        def _():
            idx_fetch(g + 2, islot ^ 2, par)

        @pl.when(g >= 1)
        def _():
            prev_islot = (islot + 3) % 4
            gath_drain(1 - par, prev_islot, 1 - par)
            scat_fire(1 - par, prev_islot, 1 - par)

    idx_fetch(0, 0, 0)
    idx_fetch(1, 1, 1)

    def loop(g2, _):
        step(2 * g2, g2, 0)
        step(2 * g2 + 1, g2, 1)
        return 0
    lax.fori_loop(0, n_chunks // 2, loop, 0)

    li = (n_chunks - 1) % 4
    gath_drain(1, li, 1)
    scat_fire(1, li, 1)
    scat_drain(0, li, 0)
    scat_drain(1, li, 1)


_AGG_SCRATCH = [
    pltpu.VMEM((4, KA, 128), jnp.int32),
    pltpu.VMEM((4, KA, 128), jnp.int32),
    pltpu.VMEM((2, KA, 128, 16), jnp.float32),
    pltpu.VMEM((128, 16), jnp.float32),
    pltpu.VMEM_SHARED((NP, 16), jnp.float32),
    pltpu.SemaphoreType.DMA,
    pltpu.SemaphoreType.DMA,
    pltpu.SemaphoreType.DMA,
    pltpu.SemaphoreType.DMA,
    pltpu.SemaphoreType.DMA,
    pltpu.SemaphoreType.DMA,
    pltpu.SemaphoreType.DMA,
]


@functools.cache
def _sc_agg1_kernel():
    mesh = plsc.VectorSubcoreMesh(core_axis_name="c", subcore_axis_name="s")
    return pl.kernel(
        _sc_agg1,
        out_type=jax.ShapeDtypeStruct((NC, NP, 16), jnp.float32),
        mesh=mesh,
        compiler_params=_SC_PARAMS,
        scratch_types=list(_AGG_SCRATCH),
    )


def _sc_agg1(src_hbm, dst_hbm, y1_hbm, out_hbm, sbuf, dbuf, rbuf, zbuf, acc,
             isem0, isem1, gsem0, gsem1, ssem0, ssem1, zsem):
    c = lax.axis_index("c")
    s = lax.axis_index("s")
    _zero_fill(zbuf, 128)
    _zero_acc(zbuf, acc, s, zsem)
    plsc.subcore_barrier()

    row0 = c * RPC + s * (RPC // NS)
    _agg_pipeline(src_hbm, dst_hbm, y1_hbm, acc, sbuf, dbuf, rbuf,
                  (isem0, isem1), (gsem0, gsem1), (ssem0, ssem1),
                  row0, (RPC // NS) // KA)
    plsc.subcore_barrier()
    _writeback(acc, out_hbm, c, s)


@functools.cache
def _sc_agg2_kernel():
    mesh = plsc.VectorSubcoreMesh(core_axis_name="c", subcore_axis_name="s")
    return pl.kernel(
        _sc_agg2,
        out_type=jax.ShapeDtypeStruct((NC, NP, 16), jnp.float32),
        mesh=mesh,
        compiler_params=_SC_PARAMS,
        scratch_types=list(_AGG_SCRATCH),
    )


def _sc_agg2(srca_hbm, srcb_hbm, dst_hbm, y2pairs_hbm, out_hbm, sbuf, dbuf,
             rbuf, zbuf, acc, isem0, isem1, gsem0, gsem1, ssem0, ssem1, zsem):
    c = lax.axis_index("c")
    s = lax.axis_index("s")
    _zero_fill(zbuf, 128)
    _zero_acc(zbuf, acc, s, zsem)
    plsc.subcore_barrier()

    row0 = s * (ROWS_PAD // NS)
    n_chunks = ROWS_PAD // NS // KA

    @pl.when(c == 0)
    def _():
        _agg_pipeline(srca_hbm, dst_hbm, y2pairs_hbm, acc, sbuf, dbuf, rbuf,
                      (isem0, isem1), (gsem0, gsem1), (ssem0, ssem1),
                      row0, n_chunks)

    @pl.when(c == 1)
    def _():
        _agg_pipeline(srcb_hbm, dst_hbm, y2pairs_hbm, acc, sbuf, dbuf, rbuf,
                      (isem0, isem1), (gsem0, gsem1), (ssem0, ssem1),
                      row0, n_chunks)

    plsc.subcore_barrier()
    _writeback(acc, out_hbm, c, s)


def _tc2_body(aggp_ref, y1_ref, d16_ref, w1b_ref, b1b_ref, w2b_ref, e32_ref,
              y2p_ref):
    d16 = d16_ref[...]
    pp = (aggp_ref[0] + aggp_ref[1] + y1_ref[...]) * d16
    h1 = jnp.maximum(
        jnp.dot(pp, w1b_ref[...], preferred_element_type=jnp.float32)
        + b1b_ref[...],
        0.0,
    )
    d32 = jnp.dot(d16, e32_ref[...], preferred_element_type=jnp.float32)
    y2p_ref[...] = jnp.dot(
        h1, w2b_ref[...], preferred_element_type=jnp.float32) * d32


def _tc2(agg1p, y1p, d16p, w1blk, b1blk, w2blk, e32):
    return pl.pallas_call(
        _tc2_body,
        grid=(GRID,),
        in_specs=[
            pl.BlockSpec((2, B_TC, 128), lambda i: (0, i, 0)),
            pl.BlockSpec((B_TC, 128), lambda i: (i, 0)),
            pl.BlockSpec((B_TC, 128), lambda i: (i, 0)),
            pl.BlockSpec((128, 512), lambda i: (0, 0)),
            pl.BlockSpec((1, 512), lambda i: (0, 0)),
            pl.BlockSpec((512, 256), lambda i: (0, 0)),
            pl.BlockSpec((128, 256), lambda i: (0, 0)),
        ],
        out_specs=pl.BlockSpec((B_TC, 256), lambda i: (i, 0)),
        out_shape=jax.ShapeDtypeStruct((PRP, 256), jnp.float32),
    )(agg1p, y1p, d16p, w1blk, b1blk, w2blk, e32)


def _tc3_body(aggp_ref, y2p_ref, d16_ref, e32_ref, sa_ref, sb_ref, b2t_ref,
              wout_ref, bout_ref, out_ref, sacc_ref):
    i = pl.program_id(0)
    d32 = jnp.dot(d16_ref[...], e32_ref[...],
                  preferred_element_type=jnp.float32)
    agg32 = (jnp.dot(aggp_ref[0], sa_ref[...], preferred_element_type=jnp.float32)
             + jnp.dot(aggp_ref[1], sb_ref[...], preferred_element_type=jnp.float32))
    h2 = jnp.maximum(
        (agg32 + y2p_ref[...]) * d32 + b2t_ref[...], 0.0)
    rows = i * B_TC + lax.broadcasted_iota(jnp.int32, (B_TC, 256), 0)
    h2 = jnp.where(rows < PR, h2, 0.0)
    ps = jnp.sum(h2, axis=0, keepdims=True)

    @pl.when(i == 0)
    def _():
        sacc_ref[...] = ps

    @pl.when(i > 0)
    def _():
        sacc_ref[...] = sacc_ref[...] + ps

    @pl.when(i == GRID - 1)
    def _():
        fold = jnp.asarray(
            lax.broadcasted_iota(jnp.int32, (256, 32), 0) % 32
            == lax.broadcasted_iota(jnp.int32, (256, 32), 1),
            dtype=jnp.float32,
        )
        g = jnp.dot(sacc_ref[...] * (1.0 / N), fold,
                    preferred_element_type=jnp.float32)
        t = jnp.dot(g, wout_ref[...],
                    preferred_element_type=jnp.float32) + bout_ref[...]
        out_ref[...] = 1.0 / (1.0 + jnp.exp(-t))


def _tc3(agg2p, y2p, d16p, e32, sa, sb, b2t, wout, boutr):
    return pl.pallas_call(
        _tc3_body,
        grid=(GRID,),
        in_specs=[
            pl.BlockSpec((2, B_TC, 128), lambda i: (0, i, 0)),
            pl.BlockSpec((B_TC, 256), lambda i: (i, 0)),
            pl.BlockSpec((B_TC, 128), lambda i: (i, 0)),
            pl.BlockSpec((128, 256), lambda i: (0, 0)),
            pl.BlockSpec((128, 256), lambda i: (0, 0)),
            pl.BlockSpec((128, 256), lambda i: (0, 0)),
            pl.BlockSpec((1, 256), lambda i: (0, 0)),
            pl.BlockSpec((32, 1), lambda i: (0, 0)),
            pl.BlockSpec((1, 1), lambda i: (0, 0)),
        ],
        out_specs=pl.BlockSpec((1, 1), lambda i: (0, 0)),
        out_shape=jax.ShapeDtypeStruct((1, 1), jnp.float32),
        scratch_shapes=[pltpu.VMEM((1, 256), jnp.float32)],
    )(agg2p, y2p, d16p, e32, sa, sb, b2t, wout, boutr)


@jax.jit
def kernel(x, edge_index, W1, b1, W2, b2, Wout, bout):
    pad = EPAD - E
    src = jnp.concatenate([edge_index[0], jnp.zeros((pad,), jnp.int32)])
    dst2d = jnp.concatenate(
        [edge_index[1], jnp.full((pad,), N, jnp.int32)]).reshape(ROWS_PAD, 128)
    src2d = src.reshape(ROWS_PAD, 128)
    src2a = (src * 2).reshape(ROWS_PAD, 128)
    src2b = (src * 2 + 1).reshape(ROWS_PAD, 128)
    xp = jnp.pad(x, ((0, NP - N), (0, 16 - x.shape[1])))
    w1p = jnp.pad(W1, ((0, 16 - W1.shape[0]), (0, 0)))
    eye8 = jnp.eye(8, dtype=jnp.float32)
    w1blk = jnp.kron(eye8, w1p)
    w2blk = jnp.kron(eye8, W2)
    e32 = jnp.kron(eye8, jnp.full((16, 32), 1.0 / 16.0))
    b1blk = jnp.tile(b1, 8).reshape(1, 512)
    b2t = jnp.tile(b2, 8).reshape(1, 256)

    d16, y1 = _sc_front_kernel()(dst2d, xp)
    agg1p = _sc_agg1_kernel()(src2d, dst2d, y1)
    y2p = _tc2(
        agg1p.reshape(NC, PRP, 128), y1.reshape(PRP, 128), d16.reshape(PRP, 128),
        w1blk, b1blk, w2blk, e32)
    y2pairs = y2p.reshape(2 * NP, 16)
    agg2p = _sc_agg2_kernel()(src2a, src2b, dst2d, y2pairs)
    i16 = jnp.eye(16, dtype=jnp.float32)
    z16 = jnp.zeros((16, 16), jnp.float32)
    sa = jnp.kron(eye8, jnp.concatenate([i16, z16], axis=1))
    sb = jnp.kron(eye8, jnp.concatenate([z16, i16], axis=1))
    out = _tc3(agg2p.reshape(NC, PRP, 128), y2p, d16.reshape(PRP, 128),
               e32, sa, sb, b2t, Wout, bout.reshape(1, 1))
    return out.reshape(1)


# speedup vs baseline: 1.4773x; 1.0017x over previous
import functools

import jax
import jax.numpy as jnp
from jax import lax
from jax.experimental import pallas as pl
from jax.experimental.pallas import tpu as pltpu
from jax.experimental.pallas import tpu_sc as plsc

N = 100000
E = 1600000
NC, NS = 2, 16
K = 14
KA = 4
ROWS_PAD = 12544
EPAD = ROWS_PAD * 128
RPC = ROWS_PAD // NC
NP = 102400
NA = NP
PR = N * 16 // 128
PRP = NP * 16 // 128
B_TC = 256
GRID = PRP // B_TC

_SC_PARAMS = pltpu.CompilerParams(use_tc_tiling_on_sc=False)
_SC_PARAMS_NLP = pltpu.CompilerParams(
    use_tc_tiling_on_sc=False, needs_layout_passes=False)


def _zero_fill(zbuf, rows):
    def body(i, _):
        zbuf[i, :] = jnp.zeros((16,), jnp.float32)
        return 0
    lax.fori_loop(0, rows, body, 0)


def _zero_acc(zbuf, acc, s, zsem):
    def zc(i, _):
        pltpu.async_copy(zbuf, acc.at[pl.ds(i * 128, 128)], zsem)
        return 0
    lax.fori_loop(s * 50, (s + 1) * 50, zc, 0)

    def zw(i, _):
        pltpu.make_async_copy(zbuf, acc.at[pl.ds(0, 128)], zsem).wait()
        return 0
    lax.fori_loop(0, 50, zw, 0)


def _writeback(acc, out_hbm, c, s):
    def wb(i, _):
        off = pl.multiple_of(i * 2048, 8)
        pltpu.sync_copy(acc.at[pl.ds(off, 2048)], out_hbm.at[c, pl.ds(off, 2048)])
        return 0
    lax.fori_loop((s * 50) // 16, ((s + 1) * 50) // 16, wb, 0)


def _rsqrt16(x):
    i = lax.bitcast_convert_type(x, jnp.int32)
    i = jnp.full((16,), 0x5F3759DF, jnp.int32) - lax.shift_right_arithmetic(
        i, jnp.full((16,), 1, jnp.int32))
    y = lax.bitcast_convert_type(i, jnp.float32)
    for _ in range(4):
        y = y * (1.5 - 0.5 * x * y * y)
    return y


@functools.cache
def _sc_front_kernel():
    mesh = plsc.VectorSubcoreMesh(core_axis_name="c", subcore_axis_name="s")
    return pl.kernel(
        _sc_front,
        out_type=(
            jax.ShapeDtypeStruct((NP, 16), jnp.float32),
            jax.ShapeDtypeStruct((NP, 16), jnp.float32),
        ),
        mesh=mesh,
        compiler_params=_SC_PARAMS_NLP,
        scratch_types=[
            pltpu.VMEM((2, K, 128), jnp.int32),
            pltpu.VMEM((128,), jnp.float32),
            pltpu.VMEM((2048,), jnp.float32),
            pltpu.VMEM((800,), jnp.float32),
            pltpu.VMEM((800, 16), jnp.float32),
            pltpu.VMEM((800, 16), jnp.float32),
            pltpu.VMEM_SHARED((NA,), jnp.float32),
            pltpu.SemaphoreType.DMA,
            pltpu.SemaphoreType.DMA,
            pltpu.SemaphoreType.DMA,
            pltpu.SemaphoreType.DMA,
        ],
    )


def _sc_front(dst_hbm, xp_hbm, d16_hbm, y1_hbm, dbuf, ones, zbuf, degb, d16b,
              xpb, acc, isem, ssem0, ssem1, zsem):
    c = lax.axis_index("c")
    s = lax.axis_index("s")
    ssems = (ssem0, ssem1)

    def ob(i, _):
        ones[pl.ds(i * 16, 16)] = jnp.full((16,), 1.0, jnp.float32)
        return 0
    lax.fori_loop(0, 8, ob, 0)

    def zb(i, _):
        zbuf[pl.ds(i * 16, 16)] = jnp.zeros((16,), jnp.float32)
        return 0
    lax.fori_loop(0, 128, zb, 0)

    def zc(i, _):
        pltpu.async_copy(zbuf, acc.at[pl.ds(i * 2048, 2048)], zsem)
        return 0
    lax.fori_loop((s * 50) // 16, ((s + 1) * 50) // 16, zc, 0)

    def zw(i, _):
        pltpu.make_async_copy(zbuf, acc.at[pl.ds(0, 2048)], zsem).wait()
        return 0
    lax.fori_loop((s * 50) // 16, ((s + 1) * 50) // 16, zw, 0)
    plsc.subcore_barrier()

    row0 = s * (ROWS_PAD // NS)
    n_chunks = ROWS_PAD // NS // K

    def step(g, slot):
        pltpu.make_async_copy(dst_hbm.at[pl.ds(0, K)], dbuf.at[slot], isem).wait()

        @pl.when(g + 1 < n_chunks)
        def _():
            r = row0 + (g + 1) * K
            pltpu.async_copy(dst_hbm.at[pl.ds(r, K)], dbuf.at[1 - slot], isem)

        @pl.when(g >= 2)
        def _():
            for j in range(K):
                pltpu.make_async_copy(ones, acc.at[dbuf.at[slot, j]], ssems[slot]).wait()

        for j in range(K):
            pltpu.async_copy(ones, acc.at[dbuf.at[slot, j]], ssems[slot], add=True)

    pltpu.async_copy(dst_hbm.at[pl.ds(row0, K)], dbuf.at[0], isem)

    def loop(g2, _):
        step(2 * g2, 0)
        step(2 * g2 + 1, 1)
        return 0
    lax.fori_loop(0, n_chunks // 2, loop, 0)

    for slot in (0, 1):
        for j in range(K):
            pltpu.make_async_copy(ones, acc.at[dbuf.at[slot, j]], ssems[slot]).wait()
    plsc.subcore_barrier()

    wid = c * NS + s
    lane = jnp.arange(16, dtype=jnp.int32)

    if True:
        node0 = wid * 3200

        def chunk(i, _):
            off = node0 + i * 800
            pltpu.sync_copy(acc.at[pl.ds(off, 800)], degb)
            pltpu.sync_copy(xp_hbm.at[pl.ds(off, 800)], xpb)

            def grp(g, _):
                d = degb[pl.ds(g * 16, 16)] + 1.0
                y = _rsqrt16(d)
                row_idx = lane + g * 16
                for l in range(16):
                    plsc.store_scatter(
                        d16b, [row_idx, jnp.full((16,), l, jnp.int32)], y)
                return 0
            lax.fori_loop(0, 50, grp, 0)

            def sc(n, _):
                xpb[n, :] = xpb[n, :] * d16b[n, :]
                return 0
            lax.fori_loop(0, 800, sc, 0)

            pltpu.sync_copy(d16b, d16_hbm.at[pl.ds(off, 800)])
            pltpu.sync_copy(xpb, y1_hbm.at[pl.ds(off, 800)])
            return 0
        lax.fori_loop(0, 4, chunk, 0)


def _agg_pipeline(src_hbm, dst_hbm, table, acc, sbuf, dbuf, rbuf,
                  isems, gsems, ssems, row0, n_chunks):
    EC = KA * 128

    def idx_fetch(g, islot, par):
        e0 = (row0 + g * KA) * 128
        pltpu.async_copy(src_hbm.at[pl.ds(e0, EC)], sbuf.at[islot], isems[par])
        pltpu.async_copy(dst_hbm.at[pl.ds(e0, EC)], dbuf.at[islot], isems[par])

    def idx_wait(islot, par):
        pltpu.make_async_copy(src_hbm.at[pl.ds(0, EC)], sbuf.at[islot], isems[par]).wait()
        pltpu.make_async_copy(dst_hbm.at[pl.ds(0, EC)], dbuf.at[islot], isems[par]).wait()

    def scat_fire(rslot, islot, par):
        pltpu.async_copy(rbuf.at[rslot], acc.at[dbuf.at[islot]],
                         ssems[par], add=True)

    def scat_drain(rslot, islot, par):
        pltpu.make_async_copy(rbuf.at[rslot], acc.at[dbuf.at[islot]],
                              ssems[par]).wait()

    def gath_fire(rslot, islot, par):
        pltpu.async_copy(table.at[sbuf.at[islot]], rbuf.at[rslot], gsems[par])

    def gath_drain(rslot, islot, par):
        pltpu.make_async_copy(table.at[sbuf.at[islot]], rbuf.at[rslot],
                              gsems[par]).wait()

    def step(g, g2, par):
        islot = lax.rem(g2, 2) * 2 + par
        idx_wait(islot, par)

        @pl.when(g >= 2)
        def _():
            scat_drain(par, islot, par)

        gath_fire(par, islot, par)

        @pl.when(g + 2 < n_chunks)
        def _():
            idx_fetch(g + 2, islot ^ 2, par)

        @pl.when(g >= 1)
        def _():
            prev_islot = (islot + 3) % 4
            gath_drain(1 - par, prev_islot, 1 - par)
            scat_fire(1 - par, prev_islot, 1 - par)

    idx_fetch(0, 0, 0)
    idx_fetch(1, 1, 1)

    def loop(g2, _):
        step(2 * g2, g2, 0)
        step(2 * g2 + 1, g2, 1)
        return 0
    lax.fori_loop(0, n_chunks // 2, loop, 0)

    li = (n_chunks - 1) % 4
    gath_drain(1, li, 1)
    scat_fire(1, li, 1)
    scat_drain(0, li, 0)
    scat_drain(1, li, 1)


_AGG_SCRATCH = [
    pltpu.VMEM((4, KA * 128), jnp.int32),
    pltpu.VMEM((4, KA * 128), jnp.int32),
    pltpu.VMEM((2, KA * 128, 16), jnp.float32),
    pltpu.VMEM((128, 16), jnp.float32),
    pltpu.VMEM_SHARED((NP, 16), jnp.float32),
    pltpu.SemaphoreType.DMA,
    pltpu.SemaphoreType.DMA,
    pltpu.SemaphoreType.DMA,
    pltpu.SemaphoreType.DMA,
    pltpu.SemaphoreType.DMA,
    pltpu.SemaphoreType.DMA,
    pltpu.SemaphoreType.DMA,
]


@functools.cache
def _sc_agg1_kernel():
    mesh = plsc.VectorSubcoreMesh(core_axis_name="c", subcore_axis_name="s")
    return pl.kernel(
        _sc_agg1,
        out_type=jax.ShapeDtypeStruct((NC, NP, 16), jnp.float32),
        mesh=mesh,
        compiler_params=_SC_PARAMS,
        scratch_types=list(_AGG_SCRATCH),
    )


def _sc_agg1(src_hbm, dst_hbm, y1_hbm, out_hbm, sbuf, dbuf, rbuf, zbuf, acc,
             isem0, isem1, gsem0, gsem1, ssem0, ssem1, zsem):
    c = lax.axis_index("c")
    s = lax.axis_index("s")
    _zero_fill(zbuf, 128)
    _zero_acc(zbuf, acc, s, zsem)
    plsc.subcore_barrier()

    row0 = c * RPC + s * (RPC // NS)
    _agg_pipeline(src_hbm, dst_hbm, y1_hbm, acc, sbuf, dbuf, rbuf,
                  (isem0, isem1), (gsem0, gsem1), (ssem0, ssem1),
                  row0, (RPC // NS) // KA)
    plsc.subcore_barrier()
    _writeback(acc, out_hbm, c, s)


@functools.cache
def _sc_agg2_kernel():
    mesh = plsc.VectorSubcoreMesh(core_axis_name="c", subcore_axis_name="s")
    return pl.kernel(
        _sc_agg2,
        out_type=jax.ShapeDtypeStruct((NC, NP, 16), jnp.float32),
        mesh=mesh,
        compiler_params=_SC_PARAMS,
        scratch_types=list(_AGG_SCRATCH),
    )


def _sc_agg2(srca_hbm, srcb_hbm, dst_hbm, y2pairs_hbm, out_hbm, sbuf, dbuf,
             rbuf, zbuf, acc, isem0, isem1, gsem0, gsem1, ssem0, ssem1, zsem):
    c = lax.axis_index("c")
    s = lax.axis_index("s")
    _zero_fill(zbuf, 128)
    _zero_acc(zbuf, acc, s, zsem)
    plsc.subcore_barrier()

    row0 = s * (ROWS_PAD // NS)
    n_chunks = ROWS_PAD // NS // KA

    @pl.when(c == 0)
    def _():
        _agg_pipeline(srca_hbm, dst_hbm, y2pairs_hbm, acc, sbuf, dbuf, rbuf,
                      (isem0, isem1), (gsem0, gsem1), (ssem0, ssem1),
                      row0, n_chunks)

    @pl.when(c == 1)
    def _():
        _agg_pipeline(srcb_hbm, dst_hbm, y2pairs_hbm, acc, sbuf, dbuf, rbuf,
                      (isem0, isem1), (gsem0, gsem1), (ssem0, ssem1),
                      row0, n_chunks)

    plsc.subcore_barrier()
    _writeback(acc, out_hbm, c, s)


def _tc2_body(aggp_ref, y1_ref, d16_ref, w1b_ref, b1b_ref, w2b_ref, e32_ref,
              y2p_ref):
    d16 = d16_ref[...]
    pp = (aggp_ref[0] + aggp_ref[1] + y1_ref[...]) * d16
    h1 = jnp.maximum(
        jnp.dot(pp, w1b_ref[...], preferred_element_type=jnp.float32)
        + b1b_ref[...],
        0.0,
    )
    d32 = jnp.dot(d16, e32_ref[...], preferred_element_type=jnp.float32)
    y2p_ref[...] = jnp.dot(
        h1, w2b_ref[...], preferred_element_type=jnp.float32) * d32


def _tc2(agg1p, y1p, d16p, w1blk, b1blk, w2blk, e32):
    return pl.pallas_call(
        _tc2_body,
        grid=(GRID,),
        in_specs=[
            pl.BlockSpec((2, B_TC, 128), lambda i: (0, i, 0)),
            pl.BlockSpec((B_TC, 128), lambda i: (i, 0)),
            pl.BlockSpec((B_TC, 128), lambda i: (i, 0)),
            pl.BlockSpec((128, 512), lambda i: (0, 0)),
            pl.BlockSpec((1, 512), lambda i: (0, 0)),
            pl.BlockSpec((512, 256), lambda i: (0, 0)),
            pl.BlockSpec((128, 256), lambda i: (0, 0)),
        ],
        out_specs=pl.BlockSpec((B_TC, 256), lambda i: (i, 0)),
        out_shape=jax.ShapeDtypeStruct((PRP, 256), jnp.float32),
    )(agg1p, y1p, d16p, w1blk, b1blk, w2blk, e32)


def _tc3_body(aggp_ref, y2p_ref, d16_ref, e32_ref, sa_ref, sb_ref, b2t_ref,
              wout_ref, bout_ref, out_ref, sacc_ref):
    i = pl.program_id(0)
    d32 = jnp.dot(d16_ref[...], e32_ref[...],
                  preferred_element_type=jnp.float32)
    agg32 = (jnp.dot(aggp_ref[0], sa_ref[...], preferred_element_type=jnp.float32)
             + jnp.dot(aggp_ref[1], sb_ref[...], preferred_element_type=jnp.float32))
    h2 = jnp.maximum(
        (agg32 + y2p_ref[...]) * d32 + b2t_ref[...], 0.0)
    rows = i * B_TC + lax.broadcasted_iota(jnp.int32, (B_TC, 256), 0)
    h2 = jnp.where(rows < PR, h2, 0.0)
    ps = jnp.sum(h2, axis=0, keepdims=True)

    @pl.when(i == 0)
    def _():
        sacc_ref[...] = ps

    @pl.when(i > 0)
    def _():
        sacc_ref[...] = sacc_ref[...] + ps

    @pl.when(i == GRID - 1)
    def _():
        fold = jnp.asarray(
            lax.broadcasted_iota(jnp.int32, (256, 32), 0) % 32
            == lax.broadcasted_iota(jnp.int32, (256, 32), 1),
            dtype=jnp.float32,
        )
        g = jnp.dot(sacc_ref[...] * (1.0 / N), fold,
                    preferred_element_type=jnp.float32)
        t = jnp.dot(g, wout_ref[...],
                    preferred_element_type=jnp.float32) + bout_ref[...]
        out_ref[...] = 1.0 / (1.0 + jnp.exp(-t))


def _tc3(agg2p, y2p, d16p, e32, sa, sb, b2t, wout, boutr):
    return pl.pallas_call(
        _tc3_body,
        grid=(GRID,),
        in_specs=[
            pl.BlockSpec((2, B_TC, 128), lambda i: (0, i, 0)),
            pl.BlockSpec((B_TC, 256), lambda i: (i, 0)),
            pl.BlockSpec((B_TC, 128), lambda i: (i, 0)),
            pl.BlockSpec((128, 256), lambda i: (0, 0)),
            pl.BlockSpec((128, 256), lambda i: (0, 0)),
            pl.BlockSpec((128, 256), lambda i: (0, 0)),
            pl.BlockSpec((1, 256), lambda i: (0, 0)),
            pl.BlockSpec((32, 1), lambda i: (0, 0)),
            pl.BlockSpec((1, 1), lambda i: (0, 0)),
        ],
        out_specs=pl.BlockSpec((1, 1), lambda i: (0, 0)),
        out_shape=jax.ShapeDtypeStruct((1, 1), jnp.float32),
        scratch_shapes=[pltpu.VMEM((1, 256), jnp.float32)],
    )(agg2p, y2p, d16p, e32, sa, sb, b2t, wout, boutr)


@jax.jit
def kernel(x, edge_index, W1, b1, W2, b2, Wout, bout):
    pad = EPAD - E
    src = jnp.concatenate([edge_index[0], jnp.zeros((pad,), jnp.int32)])
    dst2d = jnp.concatenate(
        [edge_index[1], jnp.full((pad,), N, jnp.int32)]).reshape(ROWS_PAD, 128)
    src2d = src.reshape(ROWS_PAD, 128)
    src2a = (src * 2).reshape(ROWS_PAD, 128)
    src2b = (src * 2 + 1).reshape(ROWS_PAD, 128)
    xp = jnp.pad(x, ((0, NP - N), (0, 16 - x.shape[1])))
    w1p = jnp.pad(W1, ((0, 16 - W1.shape[0]), (0, 0)))
    eye8 = jnp.eye(8, dtype=jnp.float32)
    w1blk = jnp.kron(eye8, w1p)
    w2blk = jnp.kron(eye8, W2)
    e32 = jnp.kron(eye8, jnp.full((16, 32), 1.0 / 16.0))
    b1blk = jnp.tile(b1, 8).reshape(1, 512)
    b2t = jnp.tile(b2, 8).reshape(1, 256)

    srcf = src2d.reshape(EPAD)
    dstf = dst2d.reshape(EPAD)

    d16, y1 = _sc_front_kernel()(dst2d, xp)
    agg1p = _sc_agg1_kernel()(srcf, dstf, y1)
    y2p = _tc2(
        agg1p.reshape(NC, PRP, 128), y1.reshape(PRP, 128), d16.reshape(PRP, 128),
        w1blk, b1blk, w2blk, e32)
    y2pairs = y2p.reshape(2 * NP, 16)
    agg2p = _sc_agg2_kernel()(src2a.reshape(EPAD), src2b.reshape(EPAD), dstf, y2pairs)
    i16 = jnp.eye(16, dtype=jnp.float32)
    z16 = jnp.zeros((16, 16), jnp.float32)
    sa = jnp.kron(eye8, jnp.concatenate([i16, z16], axis=1))
    sb = jnp.kron(eye8, jnp.concatenate([z16, i16], axis=1))
    out = _tc3(agg2p.reshape(NC, PRP, 128), y2p, d16.reshape(PRP, 128),
               e32, sa, sb, b2t, Wout, bout.reshape(1, 1))
    return out.reshape(1)
